# Initial kernel scaffold; baseline (speedup 1.0000x reference)
import jax
import jax.numpy as jnp
from jax.experimental import pallas as pl


def kernel(x, edge_index, batch, W1, b1, W2, b2, Wfc, bfc):
    raise NotImplementedError("write your pallas kernel here")


# speedup vs baseline: 36.2291x; 36.2291x over previous
import functools

import jax
import jax.numpy as jnp
from jax import lax
from jax.experimental import pallas as pl
from jax.experimental.pallas import tpu as pltpu
from jax.experimental.pallas import tpu_sc as plsc

NN = 100000
EE = 3200000
NBATCH = 64
NC = 2
NS = 16
NW = NC * NS
CH = 100
GRP = 10
ROWS = EE // CH
NPT = NN // NS
F1 = 16
F2 = 32
RB = 5000
TCG = NN // RB


def _sc_mesh():
    return plsc.VectorSubcoreMesh(
        core_axis_name="c", subcore_axis_name="s",
        num_cores=NC, num_subcores=NS)


def _agg_loop(g_hbm, src2d, dst2d, sidx, didx, rows, sem, ashared,
              base, n_groups):
    def body(i, carry):
        r0 = base + i * GRP
        pltpu.sync_copy(src2d.at[pl.ds(r0, GRP)], sidx)
        pltpu.sync_copy(dst2d.at[pl.ds(r0, GRP)], didx)
        descs = []
        for j in range(GRP):
            descs.append(
                pltpu.async_copy(g_hbm.at[sidx.at[j]], rows.at[j], sem))
        for j in range(GRP):
            descs[j].wait()
        for j in range(GRP):
            pltpu.sync_copy(rows.at[j], ashared.at[didx.at[j]], add=True)
        return carry
    lax.fori_loop(0, n_groups, body, 0)


def _make_deg_kernel():
    mesh = _sc_mesh()
    n_groups = ROWS // NW // GRP

    @functools.partial(
        pl.kernel,
        out_type=jax.ShapeDtypeStruct((NC, NN, F1), jnp.float32),
        mesh=mesh,
        compiler_params=pltpu.CompilerParams(use_tc_tiling_on_sc=False),
        scratch_types=[
            pltpu.VMEM((GRP, CH), jnp.int32),
            pltpu.VMEM((CH, F1), jnp.float32),
            pltpu.VMEM_SHARED((NN, F1), jnp.float32),
        ],
    )
    def deg_k(dst2d, ones_hbm, z_hbm, out_hbm, didx, ones_v, dshared):
        c = lax.axis_index("c")
        s = lax.axis_index("s")
        wid = s * NC + c
        pltpu.sync_copy(ones_hbm, ones_v)
        pltpu.sync_copy(z_hbm.at[pl.ds(s * NPT, NPT)],
                        dshared.at[pl.ds(s * NPT, NPT)])
        plsc.subcore_barrier()
        base = wid * (ROWS // NW)

        def body(i, carry):
            pltpu.sync_copy(dst2d.at[pl.ds(base + i * GRP, GRP)], didx)
            for j in range(GRP):
                pltpu.sync_copy(ones_v, dshared.at[didx.at[j]], add=True)
            return carry
        lax.fori_loop(0, n_groups, body, 0)
        plsc.subcore_barrier()
        pltpu.sync_copy(dshared.at[pl.ds(s * NPT, NPT)],
                        out_hbm.at[c, pl.ds(s * NPT, NPT)])

    return deg_k


def _make_agg1_kernel():
    mesh = _sc_mesh()
    rows_per_tile = ROWS // NW
    n_groups = rows_per_tile // GRP

    @functools.partial(
        pl.kernel,
        out_type=jax.ShapeDtypeStruct((NC, NN, F1), jnp.float32),
        mesh=mesh,
        compiler_params=pltpu.CompilerParams(use_tc_tiling_on_sc=False),
        scratch_types=[
            pltpu.VMEM((GRP, CH), jnp.int32),
            pltpu.VMEM((GRP, CH), jnp.int32),
            pltpu.VMEM((GRP, CH, F1), jnp.float32),
            pltpu.SemaphoreType.DMA,
            pltpu.VMEM_SHARED((NN, F1), jnp.float32),
        ],
    )
    def agg1_k(g_hbm, src2d, dst2d, z_hbm, out_hbm,
               sidx, didx, rows, sem, ashared):
        c = lax.axis_index("c")
        s = lax.axis_index("s")
        wid = s * NC + c
        pltpu.sync_copy(z_hbm.at[pl.ds(s * NPT, NPT)],
                        ashared.at[pl.ds(s * NPT, NPT)])
        plsc.subcore_barrier()
        _agg_loop(g_hbm, src2d, dst2d, sidx, didx, rows, sem, ashared,
                  wid * rows_per_tile, n_groups)
        plsc.subcore_barrier()
        pltpu.sync_copy(ashared.at[pl.ds(s * NPT, NPT)],
                        out_hbm.at[c, pl.ds(s * NPT, NPT)])

    return agg1_k


def _make_agg2_kernel():
    mesh = _sc_mesh()
    rows_per_tile = ROWS // NS
    n_groups = rows_per_tile // GRP

    @functools.partial(
        pl.kernel,
        out_type=jax.ShapeDtypeStruct((NC, NN, F1), jnp.float32),
        mesh=mesh,
        compiler_params=pltpu.CompilerParams(use_tc_tiling_on_sc=False),
        scratch_types=[
            pltpu.VMEM((GRP, CH), jnp.int32),
            pltpu.VMEM((GRP, CH), jnp.int32),
            pltpu.VMEM((GRP, CH, F1), jnp.float32),
            pltpu.SemaphoreType.DMA,
            pltpu.VMEM_SHARED((NN, F1), jnp.float32),
        ],
    )
    def agg2_k(g2a_hbm, g2b_hbm, src2d, dst2d, z_hbm, out_hbm,
               sidx, didx, rows, sem, ashared):
        c = lax.axis_index("c")
        s = lax.axis_index("s")
        pltpu.sync_copy(z_hbm.at[pl.ds(s * NPT, NPT)],
                        ashared.at[pl.ds(s * NPT, NPT)])
        plsc.subcore_barrier()
        base = s * rows_per_tile

        @pl.when(c == 0)
        def _():
            _agg_loop(g2a_hbm, src2d, dst2d, sidx, didx, rows, sem,
                      ashared, base, n_groups)

        @pl.when(c == 1)
        def _():
            _agg_loop(g2b_hbm, src2d, dst2d, sidx, didx, rows, sem,
                      ashared, base, n_groups)

        plsc.subcore_barrier()
        pltpu.sync_copy(ashared.at[pl.ds(s * NPT, NPT)],
                        out_hbm.at[c, pl.ds(s * NPT, NPT)])

    return agg2_k


def _tc_prep(deg2, x, w1):
    def body(degref, xref, w1ref, dinvref, g1ref):
        deg = degref[0] + degref[1] + 1.0
        dinv = lax.rsqrt(deg)
        h = jnp.dot(xref[...], w1ref[...],
                    preferred_element_type=jnp.float32)
        dinvref[...] = dinv
        g1ref[...] = dinv * h

    return pl.pallas_call(
        body,
        grid=(TCG,),
        in_specs=[
            pl.BlockSpec((NC, RB, F1), lambda i: (0, i, 0)),
            pl.BlockSpec((RB, 4), lambda i: (i, 0)),
            pl.BlockSpec((4, F1), lambda i: (0, 0)),
        ],
        out_specs=[
            pl.BlockSpec((RB, F1), lambda i: (i, 0)),
            pl.BlockSpec((RB, F1), lambda i: (i, 0)),
        ],
        out_shape=[
            jax.ShapeDtypeStruct((NN, F1), jnp.float32),
            jax.ShapeDtypeStruct((NN, F1), jnp.float32),
        ],
    )(deg2, x, w1)


def _tc_mid(agg1, g1, dinv16, w2, b1r):
    def body(aggref, g1ref, dinvref, w2ref, b1ref, g2aref, g2bref):
        a = aggref[0] + aggref[1] + g1ref[...]
        out1 = jnp.maximum(dinvref[...] * a + b1ref[...], 0.0)
        h2 = jnp.dot(out1, w2ref[...], preferred_element_type=jnp.float32)
        g2 = dinvref[...][:, 0:1] * h2
        g2aref[...] = g2[:, :F1]
        g2bref[...] = g2[:, F1:]

    return pl.pallas_call(
        body,
        grid=(TCG,),
        in_specs=[
            pl.BlockSpec((NC, RB, F1), lambda i: (0, i, 0)),
            pl.BlockSpec((RB, F1), lambda i: (i, 0)),
            pl.BlockSpec((RB, F1), lambda i: (i, 0)),
            pl.BlockSpec((F1, F2), lambda i: (0, 0)),
            pl.BlockSpec((1, F1), lambda i: (0, 0)),
        ],
        out_specs=[
            pl.BlockSpec((RB, F1), lambda i: (i, 0)),
            pl.BlockSpec((RB, F1), lambda i: (i, 0)),
        ],
        out_shape=[
            jax.ShapeDtypeStruct((NN, F1), jnp.float32),
            jax.ShapeDtypeStruct((NN, F1), jnp.float32),
        ],
    )(agg1, g1, dinv16, w2, b1r)


def _tc_final(agg2, g2a, g2b, dinv16, batch16, b2r, wfc, bfcr):
    def body(aggref, g2aref, g2bref, dinvref, batchref, b2ref, wfcref,
             bfcref, outref, accref):
        i = pl.program_id(0)
        dinv = dinvref[...]
        b2 = b2ref[...]
        o2a = jnp.maximum(dinv * (aggref[0] + g2aref[...]) + b2[:, :F1],
                          0.0)
        o2b = jnp.maximum(dinv * (aggref[1] + g2bref[...]) + b2[:, F1:],
                          0.0)
        out2 = jnp.concatenate([o2a, o2b], axis=1)
        feat = jnp.concatenate(
            [out2, jnp.ones((RB, 1), jnp.float32)], axis=1)
        bcol = batchref[...][:, 0:1]
        onehot = (lax.broadcasted_iota(jnp.int32, (RB, NBATCH), 1)
                  == bcol).astype(jnp.float32)
        part = lax.dot_general(onehot, feat, (((0,), (0,)), ((), ())),
                               preferred_element_type=jnp.float32)

        @pl.when(i == 0)
        def _():
            accref[...] = jnp.zeros_like(accref)

        accref[...] += part

        @pl.when(i == TCG - 1)
        def _():
            a = accref[...]
            pooled = a[:, :F2] / jnp.maximum(a[:, F2:F2 + 1], 1.0)
            z = jnp.dot(pooled, wfcref[...],
                        preferred_element_type=jnp.float32) + bfcref[...]
            outref[...] = jax.nn.sigmoid(z)

    return pl.pallas_call(
        body,
        grid=(TCG,),
        in_specs=[
            pl.BlockSpec((NC, RB, F1), lambda i: (0, i, 0)),
            pl.BlockSpec((RB, F1), lambda i: (i, 0)),
            pl.BlockSpec((RB, F1), lambda i: (i, 0)),
            pl.BlockSpec((RB, F1), lambda i: (i, 0)),
            pl.BlockSpec((RB, F1), lambda i: (i, 0)),
            pl.BlockSpec((1, F2), lambda i: (0, 0)),
            pl.BlockSpec((F2, 1), lambda i: (0, 0)),
            pl.BlockSpec((1, 1), lambda i: (0, 0)),
        ],
        out_specs=pl.BlockSpec((NBATCH, 1), lambda i: (0, 0)),
        out_shape=jax.ShapeDtypeStruct((NBATCH, 1), jnp.float32),
        scratch_shapes=[pltpu.VMEM((NBATCH, F2 + 1), jnp.float32)],
    )(agg2, g2a, g2b, dinv16, batch16, b2r, wfc, bfcr)


def kernel(x, edge_index, batch, W1, b1, W2, b2, Wfc, bfc):
    src2d = edge_index[0].reshape(ROWS, CH)
    dst2d = edge_index[1].reshape(ROWS, CH)
    z16 = jnp.zeros((NN, F1), jnp.float32)
    ones_rows = jnp.ones((CH, F1), jnp.float32)
    batch16 = jnp.broadcast_to(batch[:, None], (NN, F1))
    b1r = b1.reshape(1, F1)
    b2r = b2.reshape(1, F2)
    bfcr = bfc.reshape(1, 1)

    deg2 = _make_deg_kernel()(dst2d, ones_rows, z16)
    dinv16, g1 = _tc_prep(deg2, x, W1)
    agg1 = _make_agg1_kernel()(g1, src2d, dst2d, z16)
    g2a, g2b = _tc_mid(agg1, g1, dinv16, W2, b1r)
    agg2 = _make_agg2_kernel()(g2a, g2b, src2d, dst2d, z16)
    return _tc_final(agg2, g2a, g2b, dinv16, batch16, b2r, Wfc, bfcr)


# speedup vs baseline: 48.6942x; 1.3441x over previous
import functools

import jax
import jax.numpy as jnp
from jax import lax
from jax.experimental import pallas as pl
from jax.experimental.pallas import tpu as pltpu
from jax.experimental.pallas import tpu_sc as plsc

NN = 100000
EE = 3200000
NBATCH = 64
NC = 2
NS = 16
NW = NC * NS
GC = 625
GROUPS = EE // GC
NPT = NN // NS
F1 = 16
F2 = 32
RB = 5000
TCG = NN // RB


def _sc_mesh():
    return plsc.VectorSubcoreMesh(
        core_axis_name="c", subcore_axis_name="s",
        num_cores=NC, num_subcores=NS)


def _agg_loop(g_hbm, src2d, dst2d, sidx, didx, rows, semg, sems,
              ashared, base, n_groups):
    def stage(g, b):
        pltpu.sync_copy(src2d.at[base + g], sidx.at[b])
        pltpu.sync_copy(dst2d.at[base + g], didx.at[b])

    def fire_gather(b):
        pltpu.async_copy(g_hbm.at[sidx.at[b]], rows.at[b], semg[b])

    def wait_gather(b):
        pltpu.make_async_copy(g_hbm.at[sidx.at[b]], rows.at[b],
                              semg[b]).wait()

    def fire_scatter(b):
        pltpu.async_copy(rows.at[b], ashared.at[didx.at[b]], sems[b],
                         add=True)

    def wait_scatter(b):
        pltpu.make_async_copy(rows.at[b], ashared.at[didx.at[b]],
                              sems[b]).wait()

    npairs = n_groups // 2
    stage(0, 0)
    fire_gather(0)

    def body(p, carry):
        g0 = 2 * p

        @pl.when(p >= 1)
        def _():
            wait_scatter(1)
        stage(g0 + 1, 1)
        fire_gather(1)
        wait_gather(0)
        fire_scatter(0)

        @pl.when(p + 1 < npairs)
        def _():
            wait_scatter(0)
            stage(g0 + 2, 0)
            fire_gather(0)
        wait_gather(1)
        fire_scatter(1)
        return carry
    lax.fori_loop(0, npairs, body, 0)
    wait_scatter(0)
    wait_scatter(1)


def _make_deg_kernel():
    mesh = _sc_mesh()
    n_groups = GROUPS // NW

    @functools.partial(
        pl.kernel,
        out_type=jax.ShapeDtypeStruct((NC, NN, F1), jnp.float32),
        mesh=mesh,
        compiler_params=pltpu.CompilerParams(use_tc_tiling_on_sc=False),
        scratch_types=[
            pltpu.VMEM((2, GC), jnp.int32),
            pltpu.VMEM((GC, F1), jnp.float32),
            pltpu.SemaphoreType.DMA,
            pltpu.SemaphoreType.DMA,
            pltpu.VMEM_SHARED((NN, F1), jnp.float32),
        ],
    )
    def deg_k(dst2d, ones_hbm, z_hbm, out_hbm, didx, ones_v, sem0, sem1,
              dshared):
        c = lax.axis_index("c")
        s = lax.axis_index("s")
        wid = s * NC + c
        sems = (sem0, sem1)
        pltpu.sync_copy(ones_hbm, ones_v)
        pltpu.sync_copy(z_hbm.at[pl.ds(s * NPT, NPT)],
                        dshared.at[pl.ds(s * NPT, NPT)])
        plsc.subcore_barrier()
        base = wid * (GROUPS // NW)

        def stage(g, b):
            pltpu.sync_copy(dst2d.at[base + g], didx.at[b])

        def fire(b):
            pltpu.async_copy(ones_v, dshared.at[didx.at[b]], sems[b],
                             add=True)

        def wait(b):
            pltpu.make_async_copy(ones_v, dshared.at[didx.at[b]],
                                  sems[b]).wait()

        npairs = n_groups // 2
        stage(0, 0)

        def body(p, carry):
            g0 = 2 * p
            fire(0)

            @pl.when(p >= 1)
            def _():
                wait(1)
            stage(g0 + 1, 1)
            fire(1)

            @pl.when(p + 1 < npairs)
            def _():
                wait(0)
                stage(g0 + 2, 0)
            return carry
        lax.fori_loop(0, npairs, body, 0)
        wait(0)
        wait(1)
        plsc.subcore_barrier()
        pltpu.sync_copy(dshared.at[pl.ds(s * NPT, NPT)],
                        out_hbm.at[c, pl.ds(s * NPT, NPT)])

    return deg_k


def _agg_scratch():
    return [
        pltpu.VMEM((2, GC), jnp.int32),
        pltpu.VMEM((2, GC), jnp.int32),
        pltpu.VMEM((2, GC, F1), jnp.float32),
        pltpu.SemaphoreType.DMA,
        pltpu.SemaphoreType.DMA,
        pltpu.SemaphoreType.DMA,
        pltpu.SemaphoreType.DMA,
        pltpu.VMEM_SHARED((NN, F1), jnp.float32),
    ]


def _make_agg1_kernel():
    mesh = _sc_mesh()
    n_groups = GROUPS // NW

    @functools.partial(
        pl.kernel,
        out_type=jax.ShapeDtypeStruct((NC, NN, F1), jnp.float32),
        mesh=mesh,
        compiler_params=pltpu.CompilerParams(use_tc_tiling_on_sc=False),
        scratch_types=_agg_scratch(),
    )
    def agg1_k(g_hbm, src2d, dst2d, z_hbm, out_hbm,
               sidx, didx, rows, sg0, sg1, ss0, ss1, ashared):
        c = lax.axis_index("c")
        s = lax.axis_index("s")
        wid = s * NC + c
        pltpu.sync_copy(z_hbm.at[pl.ds(s * NPT, NPT)],
                        ashared.at[pl.ds(s * NPT, NPT)])
        plsc.subcore_barrier()
        _agg_loop(g_hbm, src2d, dst2d, sidx, didx, rows,
                  (sg0, sg1), (ss0, ss1), ashared,
                  wid * n_groups, n_groups)
        plsc.subcore_barrier()
        pltpu.sync_copy(ashared.at[pl.ds(s * NPT, NPT)],
                        out_hbm.at[c, pl.ds(s * NPT, NPT)])

    return agg1_k


def _make_agg2_kernel():
    mesh = _sc_mesh()
    n_groups = GROUPS // NS

    @functools.partial(
        pl.kernel,
        out_type=jax.ShapeDtypeStruct((NC, NN, F1), jnp.float32),
        mesh=mesh,
        compiler_params=pltpu.CompilerParams(use_tc_tiling_on_sc=False),
        scratch_types=_agg_scratch(),
    )
    def agg2_k(g2a_hbm, g2b_hbm, src2d, dst2d, z_hbm, out_hbm,
               sidx, didx, rows, sg0, sg1, ss0, ss1, ashared):
        c = lax.axis_index("c")
        s = lax.axis_index("s")
        pltpu.sync_copy(z_hbm.at[pl.ds(s * NPT, NPT)],
                        ashared.at[pl.ds(s * NPT, NPT)])
        plsc.subcore_barrier()
        base = s * n_groups

        @pl.when(c == 0)
        def _():
            _agg_loop(g2a_hbm, src2d, dst2d, sidx, didx, rows,
                      (sg0, sg1), (ss0, ss1), ashared, base, n_groups)

        @pl.when(c == 1)
        def _():
            _agg_loop(g2b_hbm, src2d, dst2d, sidx, didx, rows,
                      (sg0, sg1), (ss0, ss1), ashared, base, n_groups)

        plsc.subcore_barrier()
        pltpu.sync_copy(ashared.at[pl.ds(s * NPT, NPT)],
                        out_hbm.at[c, pl.ds(s * NPT, NPT)])

    return agg2_k


def _tc_prep(deg2, x, w1):
    def body(degref, xref, w1ref, dinvref, g1ref):
        deg = degref[0] + degref[1] + 1.0
        dinv = lax.rsqrt(deg)
        h = jnp.dot(xref[...], w1ref[...],
                    preferred_element_type=jnp.float32)
        dinvref[...] = dinv
        g1ref[...] = dinv * h

    return pl.pallas_call(
        body,
        grid=(TCG,),
        in_specs=[
            pl.BlockSpec((NC, RB, F1), lambda i: (0, i, 0)),
            pl.BlockSpec((RB, 4), lambda i: (i, 0)),
            pl.BlockSpec((4, F1), lambda i: (0, 0)),
        ],
        out_specs=[
            pl.BlockSpec((RB, F1), lambda i: (i, 0)),
            pl.BlockSpec((RB, F1), lambda i: (i, 0)),
        ],
        out_shape=[
            jax.ShapeDtypeStruct((NN, F1), jnp.float32),
            jax.ShapeDtypeStruct((NN, F1), jnp.float32),
        ],
    )(deg2, x, w1)


def _tc_mid(agg1, g1, dinv16, w2, b1r):
    def body(aggref, g1ref, dinvref, w2ref, b1ref, g2aref, g2bref):
        a = aggref[0] + aggref[1] + g1ref[...]
        out1 = jnp.maximum(dinvref[...] * a + b1ref[...], 0.0)
        h2 = jnp.dot(out1, w2ref[...], preferred_element_type=jnp.float32)
        g2 = dinvref[...][:, 0:1] * h2
        g2aref[...] = g2[:, :F1]
        g2bref[...] = g2[:, F1:]

    return pl.pallas_call(
        body,
        grid=(TCG,),
        in_specs=[
            pl.BlockSpec((NC, RB, F1), lambda i: (0, i, 0)),
            pl.BlockSpec((RB, F1), lambda i: (i, 0)),
            pl.BlockSpec((RB, F1), lambda i: (i, 0)),
            pl.BlockSpec((F1, F2), lambda i: (0, 0)),
            pl.BlockSpec((1, F1), lambda i: (0, 0)),
        ],
        out_specs=[
            pl.BlockSpec((RB, F1), lambda i: (i, 0)),
            pl.BlockSpec((RB, F1), lambda i: (i, 0)),
        ],
        out_shape=[
            jax.ShapeDtypeStruct((NN, F1), jnp.float32),
            jax.ShapeDtypeStruct((NN, F1), jnp.float32),
        ],
    )(agg1, g1, dinv16, w2, b1r)


def _tc_final(agg2, g2a, g2b, dinv16, batch16, b2r, wfc, bfcr):
    def body(aggref, g2aref, g2bref, dinvref, batchref, b2ref, wfcref,
             bfcref, outref, accref):
        i = pl.program_id(0)
        dinv = dinvref[...]
        b2 = b2ref[...]
        o2a = jnp.maximum(dinv * (aggref[0] + g2aref[...]) + b2[:, :F1],
                          0.0)
        o2b = jnp.maximum(dinv * (aggref[1] + g2bref[...]) + b2[:, F1:],
                          0.0)
        out2 = jnp.concatenate([o2a, o2b], axis=1)
        feat = jnp.concatenate(
            [out2, jnp.ones((RB, 1), jnp.float32)], axis=1)
        bcol = batchref[...][:, 0:1]
        onehot = (lax.broadcasted_iota(jnp.int32, (RB, NBATCH), 1)
                  == bcol).astype(jnp.float32)
        part = lax.dot_general(onehot, feat, (((0,), (0,)), ((), ())),
                               preferred_element_type=jnp.float32)

        @pl.when(i == 0)
        def _():
            accref[...] = jnp.zeros_like(accref)

        accref[...] += part

        @pl.when(i == TCG - 1)
        def _():
            a = accref[...]
            pooled = a[:, :F2] / jnp.maximum(a[:, F2:F2 + 1], 1.0)
            z = jnp.dot(pooled, wfcref[...],
                        preferred_element_type=jnp.float32) + bfcref[...]
            outref[...] = jax.nn.sigmoid(z)

    return pl.pallas_call(
        body,
        grid=(TCG,),
        in_specs=[
            pl.BlockSpec((NC, RB, F1), lambda i: (0, i, 0)),
            pl.BlockSpec((RB, F1), lambda i: (i, 0)),
            pl.BlockSpec((RB, F1), lambda i: (i, 0)),
            pl.BlockSpec((RB, F1), lambda i: (i, 0)),
            pl.BlockSpec((RB, F1), lambda i: (i, 0)),
            pl.BlockSpec((1, F2), lambda i: (0, 0)),
            pl.BlockSpec((F2, 1), lambda i: (0, 0)),
            pl.BlockSpec((1, 1), lambda i: (0, 0)),
        ],
        out_specs=pl.BlockSpec((NBATCH, 1), lambda i: (0, 0)),
        out_shape=jax.ShapeDtypeStruct((NBATCH, 1), jnp.float32),
        scratch_shapes=[pltpu.VMEM((NBATCH, F2 + 1), jnp.float32)],
    )(agg2, g2a, g2b, dinv16, batch16, b2r, wfc, bfcr)


def kernel(x, edge_index, batch, W1, b1, W2, b2, Wfc, bfc):
    src2d = edge_index[0].reshape(GROUPS, GC)
    dst2d = edge_index[1].reshape(GROUPS, GC)
    z16 = jnp.zeros((NN, F1), jnp.float32)
    ones_rows = jnp.ones((GC, F1), jnp.float32)
    batch16 = jnp.broadcast_to(batch[:, None], (NN, F1))
    b1r = b1.reshape(1, F1)
    b2r = b2.reshape(1, F2)
    bfcr = bfc.reshape(1, 1)

    deg2 = _make_deg_kernel()(dst2d, ones_rows, z16)
    dinv16, g1 = _tc_prep(deg2, x, W1)
    agg1 = _make_agg1_kernel()(g1, src2d, dst2d, z16)
    g2a, g2b = _tc_mid(agg1, g1, dinv16, W2, b1r)
    agg2 = _make_agg2_kernel()(g2a, g2b, src2d, dst2d, z16)
    return _tc_final(agg2, g2a, g2b, dinv16, batch16, b2r, Wfc, bfcr)


# speedup vs baseline: 91.6622x; 1.8824x over previous
import functools

import jax
import jax.numpy as jnp
from jax import lax
from jax.experimental import pallas as pl
from jax.experimental.pallas import tpu as pltpu
from jax.experimental.pallas import tpu_sc as plsc

NN = 100000
NP = 100032
EE = 3200000
NBATCH = 64
NC = 2
NS = 16
NW = NC * NS
GC = 800
NPT = NP // NS
F1 = 16
F2 = 32
R128 = NP * F1 // 128


def _sc_mesh():
    return plsc.VectorSubcoreMesh(
        core_axis_name="c", subcore_axis_name="s",
        num_cores=NC, num_subcores=NS)


def _agg_loop(g_hbm, ei, sidx, didx, rows, semg, sems, ashared,
              ebase, n_groups):
    def stage(g, b):
        pltpu.sync_copy(ei.at[0, pl.ds(ebase + g * GC, GC)], sidx.at[b])
        pltpu.sync_copy(ei.at[1, pl.ds(ebase + g * GC, GC)], didx.at[b])

    def fire_gather(b):
        pltpu.async_copy(g_hbm.at[sidx.at[b]], rows.at[b], semg[b])

    def wait_gather(b):
        pltpu.make_async_copy(g_hbm.at[sidx.at[b]], rows.at[b],
                              semg[b]).wait()

    def fire_scatter(b):
        pltpu.async_copy(rows.at[b], ashared.at[didx.at[b]], sems[b],
                         add=True)

    def wait_scatter(b):
        pltpu.make_async_copy(rows.at[b], ashared.at[didx.at[b]],
                              sems[b]).wait()

    npairs = n_groups // 2
    stage(0, 0)
    fire_gather(0)

    def body(p, carry):
        g0 = 2 * p

        @pl.when(p >= 1)
        def _():
            wait_scatter(1)
        stage(g0 + 1, 1)
        fire_gather(1)
        wait_gather(0)
        fire_scatter(0)

        @pl.when(p + 1 < npairs)
        def _():
            wait_scatter(0)
            stage(g0 + 2, 0)
            fire_gather(0)
        wait_gather(1)
        fire_scatter(1)
        return carry
    lax.fori_loop(0, npairs, body, 0)
    if n_groups % 2:
        wait_scatter(0)
        stage(n_groups - 1, 0)
        fire_gather(0)
        wait_gather(0)
        fire_scatter(0)
    wait_scatter(0)
    wait_scatter(1)


def _make_deg_kernel():
    mesh = _sc_mesh()
    n_groups = EE // GC // NW

    @functools.partial(
        pl.kernel,
        out_type=jax.ShapeDtypeStruct((NC, NP, F1), jnp.float32),
        mesh=mesh,
        compiler_params=pltpu.CompilerParams(use_tc_tiling_on_sc=False),
        scratch_types=[
            pltpu.VMEM((2, GC), jnp.int32),
            pltpu.VMEM((GC, F1), jnp.float32),
            pltpu.SemaphoreType.DMA,
            pltpu.SemaphoreType.DMA,
            pltpu.VMEM_SHARED((NP, F1), jnp.float32),
        ],
    )
    def deg_k(ei, ones_hbm, z_hbm, out_hbm, didx, ones_v, sem0, sem1,
              dshared):
        c = lax.axis_index("c")
        s = lax.axis_index("s")
        wid = s * NC + c
        sems = (sem0, sem1)
        pltpu.sync_copy(ones_hbm, ones_v)
        pltpu.sync_copy(z_hbm.at[pl.ds(s * NPT, NPT)],
                        dshared.at[pl.ds(s * NPT, NPT)])
        plsc.subcore_barrier()
        ebase = wid * (EE // NW)

        def stage(g, b):
            pltpu.sync_copy(ei.at[1, pl.ds(ebase + g * GC, GC)],
                            didx.at[b])

        def fire(b):
            pltpu.async_copy(ones_v, dshared.at[didx.at[b]], sems[b],
                             add=True)

        def wait(b):
            pltpu.make_async_copy(ones_v, dshared.at[didx.at[b]],
                                  sems[b]).wait()

        npairs = n_groups // 2
        stage(0, 0)

        def body(p, carry):
            g0 = 2 * p
            fire(0)

            @pl.when(p >= 1)
            def _():
                wait(1)
            stage(g0 + 1, 1)
            fire(1)

            @pl.when(p + 1 < npairs)
            def _():
                wait(0)
                stage(g0 + 2, 0)
            return carry
        lax.fori_loop(0, npairs, body, 0)
        if n_groups % 2:
            wait(0)
            stage(n_groups - 1, 0)
            fire(0)
        wait(0)
        wait(1)
        plsc.subcore_barrier()
        pltpu.sync_copy(dshared.at[pl.ds(s * NPT, NPT)],
                        out_hbm.at[c, pl.ds(s * NPT, NPT)])

    return deg_k


def _agg_scratch():
    return [
        pltpu.VMEM((2, GC), jnp.int32),
        pltpu.VMEM((2, GC), jnp.int32),
        pltpu.VMEM((2, GC, F1), jnp.float32),
        pltpu.SemaphoreType.DMA,
        pltpu.SemaphoreType.DMA,
        pltpu.SemaphoreType.DMA,
        pltpu.SemaphoreType.DMA,
        pltpu.VMEM_SHARED((NP, F1), jnp.float32),
    ]


def _make_agg1_kernel():
    mesh = _sc_mesh()
    n_groups = EE // GC // NW

    @functools.partial(
        pl.kernel,
        out_type=jax.ShapeDtypeStruct((NC, NP, F1), jnp.float32),
        mesh=mesh,
        compiler_params=pltpu.CompilerParams(use_tc_tiling_on_sc=False),
        scratch_types=_agg_scratch(),
    )
    def agg1_k(g_hbm, ei, z_hbm, out_hbm,
               sidx, didx, rows, sg0, sg1, ss0, ss1, ashared):
        c = lax.axis_index("c")
        s = lax.axis_index("s")
        wid = s * NC + c
        pltpu.sync_copy(z_hbm.at[pl.ds(s * NPT, NPT)],
                        ashared.at[pl.ds(s * NPT, NPT)])
        plsc.subcore_barrier()
        _agg_loop(g_hbm, ei, sidx, didx, rows,
                  (sg0, sg1), (ss0, ss1), ashared,
                  wid * (EE // NW), n_groups)
        plsc.subcore_barrier()
        pltpu.sync_copy(ashared.at[pl.ds(s * NPT, NPT)],
                        out_hbm.at[c, pl.ds(s * NPT, NPT)])

    return agg1_k


def _make_agg2_kernel():
    mesh = _sc_mesh()
    n_groups = EE // GC // NS

    @functools.partial(
        pl.kernel,
        out_type=jax.ShapeDtypeStruct((NC, NP, F1), jnp.float32),
        mesh=mesh,
        compiler_params=pltpu.CompilerParams(use_tc_tiling_on_sc=False),
        scratch_types=_agg_scratch(),
    )
    def agg2_k(g2a_hbm, g2b_hbm, ei, z_hbm, out_hbm,
               sidx, didx, rows, sg0, sg1, ss0, ss1, ashared):
        c = lax.axis_index("c")
        s = lax.axis_index("s")
        pltpu.sync_copy(z_hbm.at[pl.ds(s * NPT, NPT)],
                        ashared.at[pl.ds(s * NPT, NPT)])
        plsc.subcore_barrier()
        ebase = s * (EE // NS)

        @pl.when(c == 0)
        def _():
            _agg_loop(g2a_hbm, ei, sidx, didx, rows,
                      (sg0, sg1), (ss0, ss1), ashared, ebase, n_groups)

        @pl.when(c == 1)
        def _():
            _agg_loop(g2b_hbm, ei, sidx, didx, rows,
                      (sg0, sg1), (ss0, ss1), ashared, ebase, n_groups)

        plsc.subcore_barrier()
        pltpu.sync_copy(ashared.at[pl.ds(s * NPT, NPT)],
                        out_hbm.at[c, pl.ds(s * NPT, NPT)])

    return agg2_k


_TC_PARAMS = pltpu.CompilerParams(vmem_limit_bytes=110 * 1024 * 1024)


def _full(shape):
    n = len(shape)
    return pl.BlockSpec(shape, lambda: (0,) * n)


def _tc_h1(xpv, wk1):
    def body(xref, wref, href):
        href[...] = jnp.dot(xref[...], wref[...],
                            preferred_element_type=jnp.float32)

    return pl.pallas_call(
        body,
        in_specs=[_full((R128, 128)), _full((128, 128))],
        out_specs=_full((R128, 128)),
        out_shape=jax.ShapeDtypeStruct((R128, 128), jnp.float32),
        compiler_params=_TC_PARAMS,
    )(xpv, wk1)


def _tc_scale(deg2v, h1v):
    def body(degref, href, dinvref, g1ref):
        dinv = lax.rsqrt(degref[0] + degref[1] + 1.0)
        dinvref[...] = dinv
        g1ref[...] = dinv * href[...]

    return pl.pallas_call(
        body,
        in_specs=[_full((NC, R128, 128)), _full((R128, 128))],
        out_specs=[_full((R128, 128)), _full((R128, 128))],
        out_shape=[
            jax.ShapeDtypeStruct((R128, 128), jnp.float32),
            jax.ShapeDtypeStruct((R128, 128), jnp.float32),
        ],
        compiler_params=_TC_PARAMS,
    )(deg2v, h1v)


def _tc_mid(agg1v, g1v, dinvv, wka, wkb, b1t):
    def body(aggref, g1ref, dinvref, wkaref, wkbref, b1ref,
             g2aref, g2bref):
        dinv = dinvref[...]
        o1 = jnp.maximum(
            dinv * (aggref[0] + aggref[1] + g1ref[...]) + b1ref[...],
            0.0)
        g2aref[...] = dinv * jnp.dot(o1, wkaref[...],
                                     preferred_element_type=jnp.float32)
        g2bref[...] = dinv * jnp.dot(o1, wkbref[...],
                                     preferred_element_type=jnp.float32)

    return pl.pallas_call(
        body,
        in_specs=[
            _full((NC, R128, 128)),
            _full((R128, 128)),
            _full((R128, 128)),
            _full((128, 128)),
            _full((128, 128)),
            _full((1, 128)),
        ],
        out_specs=[_full((R128, 128)), _full((R128, 128))],
        out_shape=[
            jax.ShapeDtypeStruct((R128, 128), jnp.float32),
            jax.ShapeDtypeStruct((R128, 128), jnp.float32),
        ],
        compiler_params=_TC_PARAMS,
    )(agg1v, g1v, dinvv, wka, wkb, b1t)


def _tc_final(agg2v, g2av, g2bv, dinvv, b2at, b2bt, st, en, st1, en1,
              wfc, bfcr):
    def body(aggref, g2aref, g2bref, dinvref, b2aref, b2bref, stref,
             enref, st1ref, en1ref, wfcref, bfcref, outref):
        dinv = dinvref[...]
        o2a = jnp.maximum(dinv * (aggref[0] + g2aref[...])
                          + b2aref[...], 0.0)
        o2b = jnp.maximum(dinv * (aggref[1] + g2bref[...])
                          + b2bref[...], 0.0)
        stv = stref[...]
        env = enref[...]
        ridx = lax.broadcasted_iota(jnp.int32, (R128, NBATCH), 0)
        dn = (((0,), (0,)), ((), ()))
        pa = jnp.zeros((NBATCH, F1), jnp.float32)
        pb = jnp.zeros((NBATCH, F1), jnp.float32)
        for k in range(8):
            gidx = ridx * 8 + k
            oh = ((gidx >= stv) & (gidx < env)).astype(jnp.float32)
            ma = lax.dot_general(oh, o2a, dn,
                                 preferred_element_type=jnp.float32)
            mb = lax.dot_general(oh, o2b, dn,
                                 preferred_element_type=jnp.float32)
            pa = pa + ma[:, 16 * k:16 * k + 16]
            pb = pb + mb[:, 16 * k:16 * k + 16]
        counts = jnp.maximum((en1ref[...] - st1ref[...])
                             .astype(jnp.float32), 1.0)
        pooled = jnp.concatenate([pa, pb], axis=1) / counts
        z = jnp.dot(pooled, wfcref[...],
                    preferred_element_type=jnp.float32) + bfcref[...]
        outref[...] = jax.nn.sigmoid(z)

    return pl.pallas_call(
        body,
        in_specs=[
            _full((NC, R128, 128)),
            _full((R128, 128)),
            _full((R128, 128)),
            _full((R128, 128)),
            _full((1, 128)),
            _full((1, 128)),
            _full((1, NBATCH)),
            _full((1, NBATCH)),
            _full((NBATCH, 1)),
            _full((NBATCH, 1)),
            _full((F2, 1)),
            _full((1, 1)),
        ],
        out_specs=_full((NBATCH, 1)),
        out_shape=jax.ShapeDtypeStruct((NBATCH, 1), jnp.float32),
        compiler_params=_TC_PARAMS,
    )(agg2v, g2av, g2bv, dinvv, b2at, b2bt, st, en, st1, en1, wfc, bfcr)


def kernel(x, edge_index, batch, W1, b1, W2, b2, Wfc, bfc):
    xp = jnp.pad(x, ((0, NP - NN), (0, F1 - 4)))
    xpv = xp.reshape(R128, 128)
    eye8 = jnp.eye(8, dtype=jnp.float32)
    w1p = jnp.pad(W1, ((0, F1 - 4), (0, 0)))
    wk1 = jnp.kron(eye8, w1p)
    wka = jnp.kron(eye8, W2[:, :F1])
    wkb = jnp.kron(eye8, W2[:, F1:])
    b1t = jnp.tile(b1, 8).reshape(1, 128)
    b2at = jnp.tile(b2[:F1], 8).reshape(1, 128)
    b2bt = jnp.tile(b2[F1:], 8).reshape(1, 128)
    qs = jnp.arange(NBATCH, dtype=batch.dtype)
    st = jnp.searchsorted(batch, qs, side="left").astype(jnp.int32)
    en = jnp.searchsorted(batch, qs, side="right").astype(jnp.int32)
    z16 = jnp.zeros((NP, F1), jnp.float32)
    ones_rows = jnp.ones((GC, F1), jnp.float32)
    bfcr = bfc.reshape(1, 1)

    deg2 = _make_deg_kernel()(edge_index, ones_rows, z16)
    h1v = _tc_h1(xpv, wk1)
    dinvv, g1v = _tc_scale(deg2.reshape(NC, R128, 128), h1v)

    g1 = g1v.reshape(NP, F1)
    agg1 = _make_agg1_kernel()(g1, edge_index, z16)
    g2av, g2bv = _tc_mid(agg1.reshape(NC, R128, 128), g1v, dinvv,
                         wka, wkb, b1t)

    agg2 = _make_agg2_kernel()(g2av.reshape(NP, F1),
                               g2bv.reshape(NP, F1), edge_index, z16)
    return _tc_final(agg2.reshape(NC, R128, 128), g2av, g2bv, dinvv,
                     b2at, b2bt,
                     st.reshape(1, NBATCH), en.reshape(1, NBATCH),
                     st.reshape(NBATCH, 1), en.reshape(NBATCH, 1),
                     Wfc, bfcr)
